# Initial kernel scaffold; baseline (speedup 1.0000x reference)
import jax
import jax.numpy as jnp
from jax.experimental import pallas as pl


def kernel(x, batch, alpha, gamma, beta):
    raise NotImplementedError("write your pallas kernel here")


# speedup vs baseline: 2.3635x; 2.3635x over previous
import functools

import jax
import jax.numpy as jnp
from jax import lax
from jax.experimental import pallas as pl
from jax.experimental.pallas import tpu as pltpu
from jax.experimental.pallas import tpu_sc as plsc

D = 128
N = 100000
G = 128
EPS_ = 1e-05
NC, NS, L = 2, 16, 16
NW = NC * NS
RPW = N // NW
NBLK = 25
BR = RPW // NBLK
BPAD = 256

_mesh = plsc.VectorSubcoreMesh(
    core_axis_name="c", subcore_axis_name="s", num_cores=NC, num_subcores=NS)


def _wid():
    return lax.axis_index("c") * NS + lax.axis_index("s")


def _stats_body(x_hbm, b_hbm, sum_hbm, sq_hbm, cnt_hbm,
                xbuf, bbuf, sum_v, sq_v, cnt_v):
    wid = _wid()
    iota = lax.iota(jnp.int32, L)
    ones = jnp.ones((L,), jnp.float32)
    zeros = jnp.zeros((L,), jnp.float32)

    def zloop(i, _):
        row = i >> 3
        col = (i & 7) * L
        sum_v[row, pl.ds(col, L)] = zeros
        sq_v[row, pl.ds(col, L)] = zeros
        return 0
    lax.fori_loop(0, G * 8, zloop, 0)

    def zcnt(i, _):
        cnt_v[i, :] = zeros
        return 0
    lax.fori_loop(0, G, zcnt, 0)

    def blk(b, _):
        r0 = (wid * NBLK + b) * BR
        base8 = pl.multiple_of((r0 >> 3) << 3, 8)
        off = r0 - base8
        pltpu.sync_copy(x_hbm.at[pl.ds(r0 * D, BR * D)], xbuf)
        pltpu.sync_copy(b_hbm.at[pl.ds(base8, BPAD)], bbuf)

        def row(n, _):
            gv = plsc.load_gather(bbuf, [jnp.full((L,), off + n, jnp.int32)])
            plsc.addupdate_scatter(cnt_v, [gv, iota], ones)
            for c in range(D // L):
                xv = xbuf[pl.ds(n * D + c * L, L)]
                colc = iota + (c * L)
                plsc.addupdate_scatter(sum_v, [gv, colc], xv)
                plsc.addupdate_scatter(sq_v, [gv, colc], xv * xv)
            return 0
        lax.fori_loop(0, BR, row, 0)
        return 0
    lax.fori_loop(0, NBLK, blk, 0)

    pltpu.sync_copy(sum_v, sum_hbm.at[wid])
    pltpu.sync_copy(sq_v, sq_hbm.at[wid])
    pltpu.sync_copy(cnt_v, cnt_hbm.at[wid])


_stats = pl.kernel(
    _stats_body,
    out_type=(
        jax.ShapeDtypeStruct((NW, G, D), jnp.float32),
        jax.ShapeDtypeStruct((NW, G, D), jnp.float32),
        jax.ShapeDtypeStruct((NW, G, L), jnp.float32),
    ),
    mesh=_mesh,
    scratch_types=[
        pltpu.VMEM((BR * D,), jnp.float32),
        pltpu.VMEM((BPAD,), jnp.int32),
        pltpu.VMEM((G, D), jnp.float32),
        pltpu.VMEM((G, D), jnp.float32),
        pltpu.VMEM((G, L), jnp.float32),
    ],
    compiler_params=pltpu.CompilerParams(needs_layout_passes=False),
)


def _params_body(sum_ref, sq_ref, cnt_ref, alpha_ref, gamma_ref, beta_ref,
                 a_ref, b_ref):
    s = jnp.sum(sum_ref[...], axis=0)
    q = jnp.sum(sq_ref[...], axis=0)
    cnt = jnp.sum(cnt_ref[...], axis=0)[:, 0:1]
    count = jnp.clip(cnt, 1.0, None)
    al = alpha_ref[0, 0]
    mean = s / count
    ex2 = q / count
    var = jnp.maximum(ex2 - (2.0 * al - al * al) * mean * mean, 0.0)
    inv = 1.0 / jnp.sqrt(var + EPS_)
    a = gamma_ref[...] * inv
    a_ref[...] = a
    b_ref[...] = beta_ref[...] - al * mean * a


def _params(sums, sqs, cnts, alpha, gamma, beta):
    return pl.pallas_call(
        _params_body,
        out_shape=(
            jax.ShapeDtypeStruct((G, D), jnp.float32),
            jax.ShapeDtypeStruct((G, D), jnp.float32),
        ),
    )(sums, sqs, cnts, alpha.reshape(1, 1), gamma.reshape(1, D),
      beta.reshape(1, D))


def _norm_body(x_hbm, b_hbm, a_hbm, bb_hbm, y_hbm,
               xbuf, bbuf, abuf, betabuf, ybuf):
    wid = _wid()
    iota = lax.iota(jnp.int32, L)
    pltpu.sync_copy(a_hbm, abuf)
    pltpu.sync_copy(bb_hbm, betabuf)

    def blk(b, _):
        r0 = (wid * NBLK + b) * BR
        base8 = pl.multiple_of((r0 >> 3) << 3, 8)
        off = r0 - base8
        pltpu.sync_copy(x_hbm.at[pl.ds(r0 * D, BR * D)], xbuf)
        pltpu.sync_copy(b_hbm.at[pl.ds(base8, BPAD)], bbuf)

        def row(n, _):
            gv = plsc.load_gather(bbuf, [jnp.full((L,), off + n, jnp.int32)])
            for c in range(D // L):
                colc = iota + (c * L)
                xv = xbuf[pl.ds(n * D + c * L, L)]
                av = plsc.load_gather(abuf, [gv, colc])
                bv = plsc.load_gather(betabuf, [gv, colc])
                ybuf[pl.ds(n * D + c * L, L)] = xv * av + bv
            return 0
        lax.fori_loop(0, BR, row, 0)
        pltpu.sync_copy(ybuf, y_hbm.at[pl.ds(r0 * D, BR * D)])
        return 0
    lax.fori_loop(0, NBLK, blk, 0)


_norm = pl.kernel(
    _norm_body,
    out_type=jax.ShapeDtypeStruct((N * D,), jnp.float32),
    mesh=_mesh,
    scratch_types=[
        pltpu.VMEM((BR * D,), jnp.float32),
        pltpu.VMEM((BPAD,), jnp.int32),
        pltpu.VMEM((G, D), jnp.float32),
        pltpu.VMEM((G, D), jnp.float32),
        pltpu.VMEM((BR * D,), jnp.float32),
    ],
    compiler_params=pltpu.CompilerParams(needs_layout_passes=False),
)


@jax.jit
def kernel(x, batch, alpha, gamma, beta):
    b32 = batch.astype(jnp.int32)
    bpad = jnp.concatenate([b32, jnp.zeros((128,), jnp.int32)])
    xf = x.reshape(N * D)
    sums, sqs, cnts = _stats(xf, bpad)
    a, bout = _params(sums, sqs, cnts, alpha, gamma, beta)
    return _norm(xf, bpad, a, bout).reshape(N, D)


# speedup vs baseline: 3.3018x; 1.3970x over previous
import jax
import jax.numpy as jnp
from jax import lax
from jax.experimental import pallas as pl
from jax.experimental.pallas import tpu as pltpu
from jax.experimental.pallas import tpu_sc as plsc

D = 128
N = 100000
G = 128
EPS_ = 1e-05
NC, NS, L = 2, 16, 16
NW = NC * NS
RPW = N // NW
NBLK = 25
BR = RPW // NBLK
BW = BR * D
BPAD = 256
NCH = D // L

_mesh = plsc.VectorSubcoreMesh(
    core_axis_name="c", subcore_axis_name="s", num_cores=NC, num_subcores=NS)
_sc_cp = pltpu.CompilerParams(needs_layout_passes=False)


def _wid():
    return lax.axis_index("c") * NS + lax.axis_index("s")


def _block_r0(wid, b):
    return (wid * NBLK + b) * BR


def _batch_base(r0):
    base8 = pl.multiple_of((r0 >> 3) << 3, 8)
    return base8, r0 - base8


def _stats_body(x_hbm, b_hbm, sum_hbm, sq_hbm, cnt_hbm,
                xbuf, bbuf, sum_v, sq_v, cnt_v, sx0, sx1):
    wid = _wid()
    zeros = jnp.zeros((L,), jnp.float32)

    def zloop(i, _):
        row = i >> 3
        col = (i & 7) * L
        sum_v[row, pl.ds(col, L)] = zeros
        sq_v[row, pl.ds(col, L)] = zeros
        return 0
    lax.fori_loop(0, (G + 1) * NCH, zloop, 0)

    def zcnt(i, _):
        cnt_v[i, :] = zeros
        return 0
    lax.fori_loop(0, G + 1, zcnt, 0)

    def start_in(b, par):
        r0 = _block_r0(wid, b)
        base8, _ = _batch_base(r0)
        sem = sx0 if par == 0 else sx1
        pltpu.async_copy(x_hbm.at[pl.ds(r0 * D, BW)],
                         xbuf.at[pl.ds(par * BW, BW)], sem)
        pltpu.async_copy(b_hbm.at[pl.ds(base8, BPAD)],
                         bbuf.at[pl.ds(par * BPAD, BPAD)], sem)

    def wait_in(b, par):
        r0 = _block_r0(wid, b)
        base8, _ = _batch_base(r0)
        sem = sx0 if par == 0 else sx1
        pltpu.make_async_copy(x_hbm.at[pl.ds(r0 * D, BW)],
                              xbuf.at[pl.ds(par * BW, BW)], sem).wait()
        pltpu.make_async_copy(b_hbm.at[pl.ds(base8, BPAD)],
                              bbuf.at[pl.ds(par * BPAD, BPAD)], sem).wait()

    start_in(0, 0)
    start_in(1, 1)

    def blk(b, carry):
        par = b & 1
        pb = par * BPAD

        @pl.when(par == 0)
        def _():
            wait_in(b, 0)

        @pl.when(par == 1)
        def _():
            wait_in(b, 1)

        r0 = _block_r0(wid, b)
        _, off = _batch_base(r0)

        def row(n, c):
            gprev, nstart, acc = c
            g = bbuf[pl.ds(pb + off + n, L)][0]
            nglob = b * BR + n

            def flush():
                for cc in range(NCH):
                    sum_v[gprev, pl.ds(cc * L, L)] = acc[cc]
                    sq_v[gprev, pl.ds(cc * L, L)] = acc[NCH + cc]
                cnt_v[gprev, :] = jnp.full((L,), nglob - nstart, jnp.float32)
                return (g, nglob, tuple(zeros for _ in range(2 * NCH)))

            def keep():
                return (gprev, nstart, acc)

            gprev, nstart, acc = lax.cond(g != gprev, flush, keep)
            base = par * BW + n * D
            sv = list(acc[:NCH])
            qv = list(acc[NCH:])
            for cc in range(NCH):
                xv = xbuf[pl.ds(base + cc * L, L)]
                sv[cc] = sv[cc] + xv
                qv[cc] = qv[cc] + xv * xv
            return (gprev, nstart, tuple(sv) + tuple(qv))

        carry = lax.fori_loop(0, BR, row, carry)

        @pl.when(jnp.logical_and(par == 0, b + 2 < NBLK))
        def _():
            start_in(b + 2, 0)

        @pl.when(jnp.logical_and(par == 1, b + 2 < NBLK))
        def _():
            start_in(b + 2, 1)

        return carry

    init = (jnp.int32(G), jnp.int32(0),
            tuple(zeros for _ in range(2 * NCH)))
    gl, nstart, acc = lax.fori_loop(0, NBLK, blk, init)
    for cc in range(NCH):
        sum_v[gl, pl.ds(cc * L, L)] = acc[cc]
        sq_v[gl, pl.ds(cc * L, L)] = acc[NCH + cc]
    cnt_v[gl, :] = jnp.full((L,), RPW - nstart, jnp.float32)

    pltpu.sync_copy(sum_v.at[pl.ds(0, G)], sum_hbm.at[wid])
    pltpu.sync_copy(sq_v.at[pl.ds(0, G)], sq_hbm.at[wid])
    pltpu.sync_copy(cnt_v.at[pl.ds(0, G)], cnt_hbm.at[wid])


_stats = pl.kernel(
    _stats_body,
    out_type=(
        jax.ShapeDtypeStruct((NW, G, D), jnp.float32),
        jax.ShapeDtypeStruct((NW, G, D), jnp.float32),
        jax.ShapeDtypeStruct((NW, G, L), jnp.float32),
    ),
    mesh=_mesh,
    scratch_types=[
        pltpu.VMEM((2 * BW,), jnp.float32),
        pltpu.VMEM((2 * BPAD,), jnp.int32),
        pltpu.VMEM((G + 1, D), jnp.float32),
        pltpu.VMEM((G + 1, D), jnp.float32),
        pltpu.VMEM((G + 1, L), jnp.float32),
        pltpu.SemaphoreType.DMA,
        pltpu.SemaphoreType.DMA,
    ],
    compiler_params=_sc_cp,
)


def _params_body(sum_ref, sq_ref, cnt_ref, alpha_ref, gamma_ref, beta_ref,
                 a_ref, b_ref):
    s = jnp.sum(sum_ref[...], axis=0)
    q = jnp.sum(sq_ref[...], axis=0)
    cnt = jnp.sum(cnt_ref[...], axis=0)[:, 0:1]
    count = jnp.clip(cnt, 1.0, None)
    al = alpha_ref[0, 0]
    mean = s / count
    ex2 = q / count
    var = jnp.maximum(ex2 - (2.0 * al - al * al) * mean * mean, 0.0)
    inv = 1.0 / jnp.sqrt(var + EPS_)
    a = gamma_ref[...] * inv
    a_ref[...] = a
    b_ref[...] = beta_ref[...] - al * mean * a


def _params(sums, sqs, cnts, alpha, gamma, beta):
    return pl.pallas_call(
        _params_body,
        out_shape=(
            jax.ShapeDtypeStruct((G, D), jnp.float32),
            jax.ShapeDtypeStruct((G, D), jnp.float32),
        ),
    )(sums, sqs, cnts, alpha.reshape(1, 1), gamma.reshape(1, D),
      beta.reshape(1, D))


def _norm_body(x_hbm, b_hbm, a_hbm, bb_hbm, y_hbm,
               xbuf, bbuf, abuf, betabuf, ybuf, sx0, sx1, sy0, sy1):
    wid = _wid()
    zeros = jnp.zeros((L,), jnp.float32)
    pltpu.sync_copy(a_hbm, abuf)
    pltpu.sync_copy(bb_hbm, betabuf)

    def start_in(b, par):
        r0 = _block_r0(wid, b)
        base8, _ = _batch_base(r0)
        sem = sx0 if par == 0 else sx1
        pltpu.async_copy(x_hbm.at[pl.ds(r0 * D, BW)],
                         xbuf.at[pl.ds(par * BW, BW)], sem)
        pltpu.async_copy(b_hbm.at[pl.ds(base8, BPAD)],
                         bbuf.at[pl.ds(par * BPAD, BPAD)], sem)

    def wait_in(b, par):
        r0 = _block_r0(wid, b)
        base8, _ = _batch_base(r0)
        sem = sx0 if par == 0 else sx1
        pltpu.make_async_copy(x_hbm.at[pl.ds(r0 * D, BW)],
                              xbuf.at[pl.ds(par * BW, BW)], sem).wait()
        pltpu.make_async_copy(b_hbm.at[pl.ds(base8, BPAD)],
                              bbuf.at[pl.ds(par * BPAD, BPAD)], sem).wait()

    def start_out(b, par):
        r0 = _block_r0(wid, b)
        sem = sy0 if par == 0 else sy1
        pltpu.async_copy(ybuf.at[pl.ds(par * BW, BW)],
                         y_hbm.at[pl.ds(r0 * D, BW)], sem)

    def wait_out(b, par):
        r0 = _block_r0(wid, b)
        sem = sy0 if par == 0 else sy1
        pltpu.make_async_copy(ybuf.at[pl.ds(par * BW, BW)],
                              y_hbm.at[pl.ds(r0 * D, BW)], sem).wait()

    start_in(0, 0)
    start_in(1, 1)

    def blk(b, carry):
        par = b & 1
        pb = par * BPAD

        @pl.when(par == 0)
        def _():
            wait_in(b, 0)

        @pl.when(par == 1)
        def _():
            wait_in(b, 1)

        @pl.when(jnp.logical_and(par == 0, b >= 2))
        def _():
            wait_out(b - 2, 0)

        @pl.when(jnp.logical_and(par == 1, b >= 2))
        def _():
            wait_out(b - 2, 1)

        r0 = _block_r0(wid, b)
        _, off = _batch_base(r0)

        def row(n, c):
            gprev, ab = c
            g = bbuf[pl.ds(pb + off + n, L)][0]

            def reload():
                return (g,) + tuple(
                    abuf[g, pl.ds(cc * L, L)] for cc in range(NCH)) + tuple(
                    betabuf[g, pl.ds(cc * L, L)] for cc in range(NCH))

            def keep():
                return (gprev,) + ab

            res = lax.cond(g != gprev, reload, keep)
            gprev, ab = res[0], res[1:]
            base = par * BW + n * D
            for cc in range(NCH):
                xv = xbuf[pl.ds(base + cc * L, L)]
                ybuf[pl.ds(base + cc * L, L)] = xv * ab[cc] + ab[NCH + cc]
            return (gprev, ab)

        carry = lax.fori_loop(0, BR, row, carry)

        @pl.when(par == 0)
        def _():
            start_out(b, 0)

        @pl.when(par == 1)
        def _():
            start_out(b, 1)

        @pl.when(jnp.logical_and(par == 0, b + 2 < NBLK))
        def _():
            start_in(b + 2, 0)

        @pl.when(jnp.logical_and(par == 1, b + 2 < NBLK))
        def _():
            start_in(b + 2, 1)

        return carry

    init = (jnp.int32(-1), tuple(zeros for _ in range(2 * NCH)))
    lax.fori_loop(0, NBLK, blk, init)
    wait_out(NBLK - 2, 1)
    wait_out(NBLK - 1, 0)


_norm = pl.kernel(
    _norm_body,
    out_type=jax.ShapeDtypeStruct((N * D,), jnp.float32),
    mesh=_mesh,
    scratch_types=[
        pltpu.VMEM((2 * BW,), jnp.float32),
        pltpu.VMEM((2 * BPAD,), jnp.int32),
        pltpu.VMEM((G, D), jnp.float32),
        pltpu.VMEM((G, D), jnp.float32),
        pltpu.VMEM((2 * BW,), jnp.float32),
        pltpu.SemaphoreType.DMA,
        pltpu.SemaphoreType.DMA,
        pltpu.SemaphoreType.DMA,
        pltpu.SemaphoreType.DMA,
    ],
    compiler_params=_sc_cp,
)


@jax.jit
def kernel(x, batch, alpha, gamma, beta):
    b32 = batch.astype(jnp.int32)
    bpad = jnp.concatenate([b32, jnp.zeros((128,), jnp.int32)])
    xf = x.reshape(N * D)
    sums, sqs, cnts = _stats(xf, bpad)
    a, bout = _params(sums, sqs, cnts, alpha, gamma, beta)
    return _norm(xf, bpad, a, bout).reshape(N, D)
